# baseline (speedup 1.0000x reference)
import jax
import jax.numpy as jnp
from jax.experimental import pallas as pl
from jax.experimental.pallas import tpu as pltpu

NCLS = 4
CIN = 768
HH = 32
WW = 32
HID = 256
P = HH * WW
KOUT = NCLS * HID
NTAPS = 9
PAD = 33
PADDED = 1096


def _body(x_ref, wt_ref, b1_ref, w2t_ref, b2_ref, out_ref, xn_ref, acc_ref):
    t = pl.program_id(0)

    @pl.when(t == 0)
    def _init():
        x = x_ref[...]
        ss = jnp.sum(x * x, axis=1, keepdims=True)
        inv = 1.0 / jnp.maximum(jnp.sqrt(ss), 1e-12)
        xn_ref[...] = jnp.zeros((PADDED, CIN), jnp.float32)
        xn_ref[PAD:PAD + P, :] = x * inv
        acc_ref[...] = jnp.zeros((P, KOUT), jnp.float32)

    for k in range(NTAPS):
        dy = k // 3 - 1
        dx = k % 3 - 1

        @pl.when(t == k)
        def _tap(dy=dy, dx=dx):
            start = PAD + dy * WW + dx
            xs = xn_ref[start:start + P, :]
            if dx != 0:
                xcol = jax.lax.broadcasted_iota(jnp.int32, (P, 1), 0) % WW
                valid = jnp.logical_and(xcol + dx >= 0, xcol + dx < WW)
                xs = jnp.where(valid, xs, 0.0)
            acc_ref[...] += jnp.dot(xs, wt_ref[0],
                                    preferred_element_type=jnp.float32)

    @pl.when(t == NTAPS - 1)
    def _tail():
        h = jnp.maximum(acc_ref[...] + b1_ref[...], 0.0)
        logits = jnp.dot(h, w2t_ref[...],
                         preferred_element_type=jnp.float32) + b2_ref[...]
        probs = jax.nn.sigmoid(logits)
        maxp = jnp.max(probs, axis=1, keepdims=True)
        anyd = jnp.max(logits, axis=1, keepdims=True) > 0.0
        bg = jnp.where(anyd, 0.0, 1.0 - maxp)
        out_ref[...] = jnp.concatenate([bg, probs], axis=1)


def kernel(query_feat, W1, b1, W2, b2):
    xt = query_feat.reshape(CIN, P).T
    wt = jnp.transpose(W1.reshape(KOUT, CIN, NTAPS), (2, 1, 0))
    b1r = b1.reshape(1, KOUT)
    w2t = (jnp.eye(NCLS, dtype=jnp.float32)[:, None, :]
           * W2.reshape(NCLS, HID, 1)).reshape(KOUT, NCLS)
    b2r = b2.reshape(1, NCLS)

    out = pl.pallas_call(
        _body,
        grid=(NTAPS,),
        in_specs=[
            pl.BlockSpec((P, CIN), lambda t: (0, 0)),
            pl.BlockSpec((1, CIN, KOUT), lambda t: (t, 0, 0)),
            pl.BlockSpec((1, KOUT), lambda t: (0, 0)),
            pl.BlockSpec((KOUT, NCLS), lambda t: (0, 0)),
            pl.BlockSpec((1, NCLS), lambda t: (0, 0)),
        ],
        out_specs=pl.BlockSpec((P, 1 + NCLS), lambda t: (0, 0)),
        out_shape=jax.ShapeDtypeStruct((P, 1 + NCLS), jnp.float32),
        scratch_shapes=[
            pltpu.VMEM((PADDED, CIN), jnp.float32),
            pltpu.VMEM((P, KOUT), jnp.float32),
        ],
    )(xt, wt, b1r, w2t, b2r)

    return out.T.reshape(1, 1 + NCLS, HH, WW)
